# Initial kernel scaffold; baseline (speedup 1.0000x reference)
import jax
import jax.numpy as jnp
from jax.experimental import pallas as pl


def kernel(x_features, edge_index, W_feat, b_feat, emb, W1l, b1l, W1r, W2l, b2l, W2r):
    raise NotImplementedError("write your pallas kernel here")


# speedup vs baseline: 6.4757x; 6.4757x over previous
import functools

import jax
import jax.numpy as jnp
from jax import lax
from jax.experimental import pallas as pl
from jax.experimental.pallas import tpu as pltpu
from jax.experimental.pallas import tpu_sc as plsc

N = 10000
E = 320000
F_IN = 128
D = 128
D_OUT = 64

NC = 2
NS = 16
NW = NC * NS
EPW = E // NW
K = 80
NCHUNK = EPW // K
RPT = N // NS

_MESH = plsc.VectorSubcoreMesh(core_axis_name="c", subcore_axis_name="s")


_BLK = 1000


def _prep_body(xf, embb, Wf, bf, W1l, b1l, W1r, x_o, y1_o, r1_o):
    x = jnp.dot(xf[...], Wf[...], preferred_element_type=jnp.float32)
    x = x + bf[...] + embb[...]
    x_o[...] = x
    y1_o[...] = jnp.dot(x, W1l[...], preferred_element_type=jnp.float32)
    r1_o[...] = jnp.dot(x, W1r[...], preferred_element_type=jnp.float32) + b1l[...]


def _mid_body(aggp, degp, r1, W2r, b2l, h1_o, r2_o):
    agg = aggp[0] + aggp[1]
    deg = degp[0, :, 0] + degp[1, :, 0]
    rdeg = 1.0 / jnp.clip(deg, 1.0, None)
    h1 = jnp.maximum(agg * rdeg[:, None] + r1[...], 0.0)
    h1_o[...] = h1
    r2_o[...] = jnp.dot(h1, W2r[...], preferred_element_type=jnp.float32) + b2l[...]


def _final_body(aggp, degp, r2, W2l, out_o):
    agg = aggp[0] + aggp[1]
    deg = degp[0, :, 0] + degp[1, :, 0]
    rdeg = 1.0 / jnp.clip(deg, 1.0, None)
    mean = agg * rdeg[:, None]
    out_o[...] = jnp.maximum(
        jnp.dot(mean, W2l[...], preferred_element_type=jnp.float32) + r2[...],
        0.0)


def _row_spec(w):
    return pl.BlockSpec((_BLK, w), lambda i: (i, 0))


def _full_spec(*shape):
    nd = len(shape)
    return pl.BlockSpec(shape, lambda i, _nd=nd: (0,) * _nd)


def _part_spec(w):
    return pl.BlockSpec((2, _BLK, w), lambda i: (0, i, 0))


def _tc_prep(xf, emb_n, Wf, bf, W1l, b1l, W1r):
    return pl.pallas_call(
        _prep_body,
        grid=(N // _BLK,),
        in_specs=[
            _row_spec(F_IN), _row_spec(D), _full_spec(F_IN, D),
            _full_spec(1, D), _full_spec(D, D), _full_spec(1, D),
            _full_spec(D, D),
        ],
        out_specs=[_row_spec(D), _row_spec(D), _row_spec(D)],
        out_shape=[
            jax.ShapeDtypeStruct((N, D), jnp.float32),
            jax.ShapeDtypeStruct((N, D), jnp.float32),
            jax.ShapeDtypeStruct((N, D), jnp.float32),
        ],
    )(xf, emb_n, Wf, bf, W1l, b1l, W1r)


def _tc_mid(aggp, degp, r1, W2r, b2l):
    return pl.pallas_call(
        _mid_body,
        grid=(N // _BLK,),
        in_specs=[
            _part_spec(D), _part_spec(D), _row_spec(D),
            _full_spec(D, D_OUT), _full_spec(1, D_OUT),
        ],
        out_specs=[_row_spec(D), _row_spec(D_OUT)],
        out_shape=[
            jax.ShapeDtypeStruct((N, D), jnp.float32),
            jax.ShapeDtypeStruct((N, D_OUT), jnp.float32),
        ],
    )(aggp, degp, r1, W2r, b2l)


def _tc_final(aggp, degp, r2, W2l):
    return pl.pallas_call(
        _final_body,
        grid=(N // _BLK,),
        in_specs=[_part_spec(D), _part_spec(D), _row_spec(D_OUT),
                  _full_spec(D, D_OUT)],
        out_specs=_row_spec(D_OUT),
        out_shape=jax.ShapeDtypeStruct((N, D_OUT), jnp.float32),
    )(aggp, degp, r2, W2l)


def _make_sc_agg(width):
    out_type = jax.ShapeDtypeStruct((NC, NS, RPT, width), jnp.float32)
    scratch = [
        pltpu.VMEM_SHARED((N, width), jnp.float32),
        pltpu.VMEM((NCHUNK, K), jnp.int32),
        pltpu.VMEM((NCHUNK, K), jnp.int32),
        pltpu.VMEM((K, width), jnp.float32),
        pltpu.SemaphoreType.DMA,
    ]

    @functools.partial(
        pl.kernel, mesh=_MESH, out_type=out_type, scratch_types=scratch)
    def agg_kernel(y_hbm, srcr_hbm, dstr_hbm, z_hbm, agg_out,
                   acc, srcb, dstb, rows, sem):
        c = lax.axis_index("c")
        s = lax.axis_index("s")
        wid = c * NS + s
        pltpu.sync_copy(srcr_hbm.at[wid], srcb)
        pltpu.sync_copy(dstr_hbm.at[wid], dstb)
        pltpu.sync_copy(z_hbm, acc.at[pl.ds(s * RPT, RPT)])
        plsc.subcore_barrier()

        def body(j, carry):
            pltpu.async_copy(y_hbm.at[srcb.at[j]], rows, sem).wait()
            pltpu.sync_copy(rows, acc.at[dstb.at[j]], add=True)
            return carry

        lax.fori_loop(0, NCHUNK, body, 0)
        plsc.subcore_barrier()
        pltpu.sync_copy(acc.at[pl.ds(s * RPT, RPT)], agg_out.at[c, s])

    return agg_kernel


@functools.partial(
    pl.kernel, mesh=_MESH,
    out_type=jax.ShapeDtypeStruct((NC, NS, RPT, D), jnp.float32),
    scratch_types=[
        pltpu.VMEM_SHARED((N, D), jnp.float32),
        pltpu.VMEM((NCHUNK, K), jnp.int32),
        pltpu.VMEM((K, D), jnp.float32),
    ])
def _sc_deg(dstr_hbm, z16_hbm, ones_hbm, deg_out, dega, dstb, onesb):
    c = lax.axis_index("c")
    s = lax.axis_index("s")
    wid = c * NS + s
    pltpu.sync_copy(dstr_hbm.at[wid], dstb)
    pltpu.sync_copy(z16_hbm, dega.at[pl.ds(s * RPT, RPT)])
    pltpu.sync_copy(ones_hbm, onesb)
    plsc.subcore_barrier()

    def body(j, carry):
        pltpu.sync_copy(onesb, dega.at[dstb.at[j]], add=True)
        return carry

    lax.fori_loop(0, NCHUNK, body, 0)
    plsc.subcore_barrier()
    pltpu.sync_copy(dega.at[pl.ds(s * RPT, RPT)], deg_out.at[c, s])


_sc_agg = _make_sc_agg(D)


def kernel(x_features, edge_index, W_feat, b_feat, emb, W1l, b1l, W1r,
           W2l, b2l, W2r):
    src = edge_index[0].reshape(NW, NCHUNK, K)
    dst = edge_index[1].reshape(NW, NCHUNK, K)
    emb_n = emb[:N]
    z128 = jnp.zeros((RPT, D), jnp.float32)
    ones128 = jnp.ones((K, D), jnp.float32)

    degp = _sc_deg(dst, z128, ones128).reshape(NC, N, D)
    x, y1, r1 = _tc_prep(x_features, emb_n, W_feat, b_feat.reshape(1, D),
                         W1l, b1l.reshape(1, D), W1r)
    aggp1 = _sc_agg(y1, src, dst, z128).reshape(NC, N, D)
    h1, r2 = _tc_mid(aggp1, degp, r1, W2r, b2l.reshape(1, D_OUT))
    aggp2 = _sc_agg(h1, src, dst, z128).reshape(NC, N, D)
    return _tc_final(aggp2, degp, r2, W2l)


# speedup vs baseline: 13.7750x; 2.1272x over previous
import functools

import jax
import jax.numpy as jnp
from jax import lax
from jax.experimental import pallas as pl
from jax.experimental.pallas import tpu as pltpu
from jax.experimental.pallas import tpu_sc as plsc

N = 10000
E = 320000
F_IN = 128
D = 128
D_OUT = 64

NC = 2
NS = 16
NW = NC * NS
EPW = E // NW
K = 80
NCHUNK = EPW // K
RPT = N // NS

_MESH = plsc.VectorSubcoreMesh(core_axis_name="c", subcore_axis_name="s")


_BLK = 2000


def _prep_body(xf, embb, Wf, bf, W1l, b1l, W1r, y1_o, r1_o):
    x = jnp.dot(xf[...], Wf[...], preferred_element_type=jnp.float32)
    x = x + bf[...] + embb[...]
    y1_o[...] = jnp.dot(x, W1l[...], preferred_element_type=jnp.float32)
    r1_o[...] = jnp.dot(x, W1r[...], preferred_element_type=jnp.float32) + b1l[...]


def _rdeg(degp):
    deg = degp[0, :, 0] + degp[1, :, 0]
    return 1.0 / jnp.clip(deg, 1.0, None)


def _mid_body(aggp, degp, r1, W2r, b2l, h1_o, r2_o):
    agg = aggp[0] + aggp[1]
    h1 = jnp.maximum(agg * _rdeg(degp)[:, None] + r1[...], 0.0)
    h1_o[...] = h1
    r2_o[...] = jnp.dot(h1, W2r[...], preferred_element_type=jnp.float32) + b2l[...]


def _final_body(aggp, degp, r2, W2l, out_o):
    agg = aggp[0] + aggp[1]
    mean = agg * _rdeg(degp)[:, None]
    out_o[...] = jnp.maximum(
        jnp.dot(mean, W2l[...], preferred_element_type=jnp.float32) + r2[...],
        0.0)


def _row_spec(w):
    return pl.BlockSpec((_BLK, w), lambda i: (i, 0))


def _full_spec(*shape):
    nd = len(shape)
    return pl.BlockSpec(shape, lambda i, _nd=nd: (0,) * _nd)


def _part_spec(w):
    return pl.BlockSpec((2, _BLK, w), lambda i: (0, i, 0))


def _tc_prep(xf, emb, Wf, bf, W1l, b1l, W1r):
    return pl.pallas_call(
        _prep_body,
        grid=(N // _BLK,),
        in_specs=[
            _row_spec(F_IN), _row_spec(D), _full_spec(F_IN, D),
            _full_spec(1, D), _full_spec(D, D), _full_spec(1, D),
            _full_spec(D, D),
        ],
        out_specs=[_row_spec(D), _row_spec(D)],
        out_shape=[
            jax.ShapeDtypeStruct((N, D), jnp.float32),
            jax.ShapeDtypeStruct((N, D), jnp.float32),
        ],
    )(xf, emb, Wf, bf, W1l, b1l, W1r)


def _tc_mid(aggp, degp, r1, W2r, b2l):
    return pl.pallas_call(
        _mid_body,
        grid=(N // _BLK,),
        in_specs=[
            _part_spec(D), _part_spec(1), _row_spec(D),
            _full_spec(D, D_OUT), _full_spec(1, D_OUT),
        ],
        out_specs=[_row_spec(D), _row_spec(D_OUT)],
        out_shape=[
            jax.ShapeDtypeStruct((N, D), jnp.float32),
            jax.ShapeDtypeStruct((N, D_OUT), jnp.float32),
        ],
    )(aggp, degp, r1, W2r, b2l)


def _tc_final(aggp, degp, r2, W2l):
    return pl.pallas_call(
        _final_body,
        grid=(N // _BLK,),
        in_specs=[_part_spec(D), _part_spec(1), _row_spec(D_OUT),
                  _full_spec(D, D_OUT)],
        out_specs=_row_spec(D_OUT),
        out_shape=jax.ShapeDtypeStruct((N, D_OUT), jnp.float32),
    )(aggp, degp, r2, W2l)


_OUTER = (NCHUNK - 2) // 3


def _make_sc_agg(with_deg):
    out_type = [jax.ShapeDtypeStruct((NC, N, D), jnp.float32)]
    scratch = [
        pltpu.VMEM_SHARED((N, D), jnp.float32),
        pltpu.VMEM((EPW,), jnp.int32),
        pltpu.VMEM((3, K), jnp.int32),
        pltpu.VMEM((3, K, D), jnp.float32),
    ] + [pltpu.SemaphoreType.DMA] * 9
    if with_deg:
        out_type.append(jax.ShapeDtypeStruct((NC, N), jnp.float32))
        scratch += [pltpu.SemaphoreType.DMA] * 3
        scratch += [
            pltpu.VMEM_SHARED((N,), jnp.float32),
            pltpu.VMEM((K,), jnp.float32),
        ]

    @functools.partial(
        pl.kernel, mesh=_MESH, out_type=out_type, scratch_types=scratch)
    def agg_kernel(y_hbm, src_hbm, dst_hbm, z_hbm, zn_hbm, ones_hbm, *rest):
        if with_deg:
            (agg_out, deg_out, acc, srcb, dslot, rows,
             g0, g1, g2, d0, d1, d2, s0, s1, s2,
             e0, e1, e2, dega, onesb) = rest
            dgsem = (e0, e1, e2)
        else:
            (agg_out, acc, srcb, dslot, rows,
             g0, g1, g2, d0, d1, d2, s0, s1, s2) = rest
        gsem = (g0, g1, g2)
        dsem = (d0, d1, d2)
        ssem = (s0, s1, s2)
        c = lax.axis_index("c")
        s = lax.axis_index("s")
        wid = c * NS + s
        base = wid * EPW
        pltpu.sync_copy(src_hbm.at[pl.ds(base, EPW)], srcb)
        pltpu.sync_copy(z_hbm, acc.at[pl.ds(s * RPT, RPT)])
        if with_deg:
            @pl.when(s == 0)
            def _():
                pltpu.sync_copy(zn_hbm, dega)
            pltpu.sync_copy(ones_hbm, onesb)
        plsc.subcore_barrier()

        def fire(j, slot):
            pltpu.async_copy(y_hbm.at[srcb.at[pl.ds(j * K, K)]],
                             rows.at[slot], gsem[slot])
            pltpu.async_copy(dst_hbm.at[pl.ds(base + j * K, K)],
                             dslot.at[slot], dsem[slot])

        def wait_scat(slot):
            pltpu.make_async_copy(rows.at[slot], acc.at[pl.ds(0, K)],
                                  ssem[slot]).wait()
            if with_deg:
                pltpu.make_async_copy(onesb, dega.at[pl.ds(0, K)],
                                      dgsem[slot]).wait()

        def visit(j, slot, first=False):
            nslot = (slot + 2) % 3
            pltpu.make_async_copy(y_hbm.at[pl.ds(0, K)], rows.at[slot],
                                  gsem[slot]).wait()
            pltpu.make_async_copy(dst_hbm.at[pl.ds(0, K)],
                                  dslot.at[slot], dsem[slot]).wait()
            pltpu.async_copy(rows.at[slot], acc.at[dslot.at[slot]],
                             ssem[slot], add=True)
            if with_deg:
                pltpu.async_copy(onesb, dega.at[dslot.at[slot]],
                                 dgsem[slot], add=True)
            if not first:
                wait_scat(nslot)

            @pl.when(j + 2 < NCHUNK)
            def _():
                fire(j + 2, nslot)

        fire(0, 0)
        fire(1, 1)

        visit(0, 0, first=True)
        visit(1, 1)
        visit(2, 2)

        def body3(g, carry):
            for b in range(3):
                j = 3 + g * 3 + b
                visit(j, b)
            return carry

        lax.fori_loop(0, _OUTER - 1, body3, 0)
        visit(NCHUNK - 2, (NCHUNK - 2) % 3)
        visit(NCHUNK - 1, (NCHUNK - 1) % 3)
        wait_scat((NCHUNK - 1) % 3)

        plsc.subcore_barrier()

        @pl.when(s < 10)
        def _():
            pltpu.sync_copy(acc.at[pl.ds(s * 1000, 1000)],
                            agg_out.at[c, pl.ds(s * 1000, 1000)])
        if with_deg:
            @pl.when(s == 0)
            def _():
                pltpu.sync_copy(dega, deg_out.at[c])

    return agg_kernel


_sc_agg1 = _make_sc_agg(with_deg=True)
_sc_agg2 = _make_sc_agg(with_deg=False)


def kernel(x_features, edge_index, W_feat, b_feat, emb, W1l, b1l, W1r,
           W2l, b2l, W2r):
    z128 = jnp.zeros((RPT, D), jnp.float32)
    zn = jnp.zeros((N,), jnp.float32)
    ones1 = jnp.ones((K,), jnp.float32)

    y1, r1 = _tc_prep(x_features, emb, W_feat, b_feat.reshape(1, D),
                      W1l, b1l.reshape(1, D), W1r)
    src_e = edge_index[0]
    dst_e = edge_index[1]
    aggp1, degp = _sc_agg1(y1, src_e, dst_e, z128, zn, ones1)
    degp = degp.reshape(NC, N, 1)
    h1, r2 = _tc_mid(aggp1, degp, r1, W2r, b2l.reshape(1, D_OUT))
    aggp2 = _sc_agg2(h1, src_e, dst_e, z128, zn, ones1)[0]
    return _tc_final(aggp2, degp, r2, W2l)


# speedup vs baseline: 14.3413x; 1.0411x over previous
import functools

import jax
import jax.numpy as jnp
from jax import lax
from jax.experimental import pallas as pl
from jax.experimental.pallas import tpu as pltpu
from jax.experimental.pallas import tpu_sc as plsc

N = 10000
E = 320000
F_IN = 128
D = 128
D_OUT = 64

NC = 2
NS = 16
NW = NC * NS
EPW = E // NW
K = 80
NCHUNK = EPW // K
RPT = N // NS

_MESH = plsc.VectorSubcoreMesh(core_axis_name="c", subcore_axis_name="s")


_BLK = 2000


def _edges_body(ei, src_o, dst_o):
    e = ei[...]
    src_o[...] = e[0]
    dst_o[...] = e[1]


def _tc_edges(edge_index):
    return pl.pallas_call(
        _edges_body,
        out_shape=[jax.ShapeDtypeStruct((E,), jnp.int32),
                   jax.ShapeDtypeStruct((E,), jnp.int32)],
    )(edge_index)


def _prep_body(xf, embb, Wf, bf, W1l, b1l, W1r, y1_o, r1_o):
    x = jnp.dot(xf[...], Wf[...], preferred_element_type=jnp.float32)
    x = x + bf[...] + embb[...]
    y1_o[...] = jnp.dot(x, W1l[...], preferred_element_type=jnp.float32)
    r1_o[...] = jnp.dot(x, W1r[...], preferred_element_type=jnp.float32) + b1l[...]


def _rdeg(degp):
    deg = degp[0, :, 0] + degp[1, :, 0]
    return 1.0 / jnp.clip(deg, 1.0, None)


def _mid_body(aggp0, aggp1, degp, r1, W2r, b2l, h1_o, r2_o):
    agg = aggp0[...] + aggp1[...]
    h1 = jnp.maximum(agg * _rdeg(degp)[:, None] + r1[...], 0.0)
    h1_o[...] = h1
    r2_o[...] = jnp.dot(h1, W2r[...], preferred_element_type=jnp.float32) + b2l[...]


def _final_body(aggp0, aggp1, degp, r2, W2l, out_o):
    agg = aggp0[...] + aggp1[...]
    mean = agg * _rdeg(degp)[:, None]
    out_o[...] = jnp.maximum(
        jnp.dot(mean, W2l[...], preferred_element_type=jnp.float32) + r2[...],
        0.0)


def _row_spec(w):
    return pl.BlockSpec((_BLK, w), lambda i: (i, 0))


def _full_spec(*shape):
    nd = len(shape)
    return pl.BlockSpec(shape, lambda i, _nd=nd: (0,) * _nd)


def _part_spec(w):
    return pl.BlockSpec((2, _BLK, w), lambda i: (0, i, 0))


def _tc_prep(xf, emb, Wf, bf, W1l, b1l, W1r):
    return pl.pallas_call(
        _prep_body,
        grid=(N // _BLK,),
        in_specs=[
            _row_spec(F_IN), _row_spec(D), _full_spec(F_IN, D),
            _full_spec(1, D), _full_spec(D, D), _full_spec(1, D),
            _full_spec(D, D),
        ],
        out_specs=[_row_spec(D), _row_spec(D)],
        out_shape=[
            jax.ShapeDtypeStruct((N, D), jnp.float32),
            jax.ShapeDtypeStruct((N, D), jnp.float32),
        ],
    )(xf, emb, Wf, bf, W1l, b1l, W1r)


def _tc_mid(aggp0, aggp1, degp, r1, W2r, b2l):
    return pl.pallas_call(
        _mid_body,
        grid=(N // _BLK,),
        in_specs=[
            _row_spec(D), _row_spec(D), _part_spec(1), _row_spec(D),
            _full_spec(D, D_OUT), _full_spec(1, D_OUT),
        ],
        out_specs=[_row_spec(D), _row_spec(D_OUT)],
        out_shape=[
            jax.ShapeDtypeStruct((N, D), jnp.float32),
            jax.ShapeDtypeStruct((N, D_OUT), jnp.float32),
        ],
    )(aggp0, aggp1, degp, r1, W2r, b2l)


def _tc_final(aggp0, aggp1, degp, r2, W2l):
    return pl.pallas_call(
        _final_body,
        grid=(N // _BLK,),
        in_specs=[_row_spec(D), _row_spec(D), _part_spec(1),
                  _row_spec(D_OUT), _full_spec(D, D_OUT)],
        out_specs=_row_spec(D_OUT),
        out_shape=jax.ShapeDtypeStruct((N, D_OUT), jnp.float32),
    )(aggp0, aggp1, degp, r2, W2l)


_OUTER = (NCHUNK - 2) // 3


def _make_sc_agg(with_deg):
    out_type = [jax.ShapeDtypeStruct((N, D), jnp.float32),
                jax.ShapeDtypeStruct((N, D), jnp.float32)]
    scratch = [
        pltpu.VMEM_SHARED((N, D), jnp.float32),
        pltpu.VMEM((EPW,), jnp.int32),
        pltpu.VMEM((3, K), jnp.int32),
        pltpu.VMEM((3, K, D), jnp.float32),
    ] + [pltpu.SemaphoreType.DMA] * 9
    if with_deg:
        out_type.append(jax.ShapeDtypeStruct((NC, N), jnp.float32))
        scratch += [pltpu.SemaphoreType.DMA] * 3
        scratch += [
            pltpu.VMEM_SHARED((N,), jnp.float32),
            pltpu.VMEM((K,), jnp.float32),
        ]

    @functools.partial(
        pl.kernel, mesh=_MESH, out_type=out_type, scratch_types=scratch)
    def agg_kernel(y_hbm, src_hbm, dst_hbm, z_hbm, zn_hbm, ones_hbm, *rest):
        if with_deg:
            (agg_out0, agg_out1, deg_out, acc, srcb, dslot, rows,
             g0, g1, g2, d0, d1, d2, s0, s1, s2,
             e0, e1, e2, dega, onesb) = rest
            dgsem = (e0, e1, e2)
        else:
            (agg_out0, agg_out1, acc, srcb, dslot, rows,
             g0, g1, g2, d0, d1, d2, s0, s1, s2) = rest
        gsem = (g0, g1, g2)
        dsem = (d0, d1, d2)
        ssem = (s0, s1, s2)
        c = lax.axis_index("c")
        s = lax.axis_index("s")
        wid = c * NS + s
        base = wid * EPW
        pltpu.sync_copy(src_hbm.at[pl.ds(base, EPW)], srcb)
        pltpu.sync_copy(z_hbm, acc.at[pl.ds(s * RPT, RPT)])
        if with_deg:
            @pl.when(s == 0)
            def _():
                pltpu.sync_copy(zn_hbm, dega)
            pltpu.sync_copy(ones_hbm, onesb)
        plsc.subcore_barrier()

        def fire(j, slot):
            pltpu.async_copy(y_hbm.at[srcb.at[pl.ds(j * K, K)]],
                             rows.at[slot], gsem[slot])
            pltpu.async_copy(dst_hbm.at[pl.ds(base + j * K, K)],
                             dslot.at[slot], dsem[slot])

        def wait_scat(slot):
            pltpu.make_async_copy(rows.at[slot], acc.at[pl.ds(0, K)],
                                  ssem[slot]).wait()
            if with_deg:
                pltpu.make_async_copy(onesb, dega.at[pl.ds(0, K)],
                                      dgsem[slot]).wait()

        def visit(j, slot, first=False):
            nslot = (slot + 2) % 3
            pltpu.make_async_copy(y_hbm.at[pl.ds(0, K)], rows.at[slot],
                                  gsem[slot]).wait()
            pltpu.make_async_copy(dst_hbm.at[pl.ds(0, K)],
                                  dslot.at[slot], dsem[slot]).wait()
            pltpu.async_copy(rows.at[slot], acc.at[dslot.at[slot]],
                             ssem[slot], add=True)
            if with_deg:
                pltpu.async_copy(onesb, dega.at[dslot.at[slot]],
                                 dgsem[slot], add=True)
            if not first:
                wait_scat(nslot)

            @pl.when(j + 2 < NCHUNK)
            def _():
                fire(j + 2, nslot)

        fire(0, 0)
        fire(1, 1)

        visit(0, 0, first=True)
        visit(1, 1)
        visit(2, 2)

        def body3(g, carry):
            for b in range(3):
                j = 3 + g * 3 + b
                visit(j, b)
            return carry

        lax.fori_loop(0, _OUTER - 1, body3, 0)
        visit(NCHUNK - 2, (NCHUNK - 2) % 3)
        visit(NCHUNK - 1, (NCHUNK - 1) % 3)
        wait_scat((NCHUNK - 1) % 3)

        plsc.subcore_barrier()

        @pl.when((s < 10) & (c == 0))
        def _():
            pltpu.sync_copy(acc.at[pl.ds(s * 1000, 1000)],
                            agg_out0.at[pl.ds(s * 1000, 1000)])

        @pl.when((s < 10) & (c == 1))
        def _():
            pltpu.sync_copy(acc.at[pl.ds(s * 1000, 1000)],
                            agg_out1.at[pl.ds(s * 1000, 1000)])
        if with_deg:
            @pl.when(s == 0)
            def _():
                pltpu.sync_copy(dega, deg_out.at[c])

    return agg_kernel


_sc_agg1 = _make_sc_agg(with_deg=True)
_sc_agg2 = _make_sc_agg(with_deg=False)


def kernel(x_features, edge_index, W_feat, b_feat, emb, W1l, b1l, W1r,
           W2l, b2l, W2r):
    z128 = jnp.zeros((RPT, D), jnp.float32)
    zn = jnp.zeros((N,), jnp.float32)
    ones1 = jnp.ones((K,), jnp.float32)

    y1, r1 = _tc_prep(x_features, emb, W_feat, b_feat.reshape(1, D),
                      W1l, b1l.reshape(1, D), W1r)
    src_e, dst_e = _tc_edges(edge_index)
    p1a, p1b, degp = _sc_agg1(y1, src_e, dst_e, z128, zn, ones1)
    degp = degp.reshape(NC, N, 1)
    h1, r2 = _tc_mid(p1a, p1b, degp, r1, W2r, b2l.reshape(1, D_OUT))
    p2a, p2b = _sc_agg2(h1, src_e, dst_e, z128, zn, ones1)
    return _tc_final(p2a, p2b, degp, r2, W2l)


# speedup vs baseline: 14.4634x; 1.0085x over previous
import functools

import jax
import jax.numpy as jnp
from jax import lax
from jax.experimental import pallas as pl
from jax.experimental.pallas import tpu as pltpu
from jax.experimental.pallas import tpu_sc as plsc

N = 10000
E = 320000
F_IN = 128
D = 128
D_OUT = 64

NC = 2
NS = 16
NW = NC * NS
EPW = E // NW
K = 80
NCHUNK = EPW // K
RPT = N // NS

_MESH = plsc.VectorSubcoreMesh(core_axis_name="c", subcore_axis_name="s")


_BLK = 2000


def _edges_body(ei, src_o, dst_o):
    e = ei[...]
    src_o[...] = e[0]
    dst_o[...] = e[1]


def _tc_edges(edge_index):
    return pl.pallas_call(
        _edges_body,
        out_shape=[jax.ShapeDtypeStruct((E,), jnp.int32),
                   jax.ShapeDtypeStruct((E,), jnp.int32)],
    )(edge_index)


def _prep_body(xf, embb, Wf, bf, W1l, b1l, W1r, y1_o, r1_o):
    x = jnp.dot(xf[...], Wf[...], preferred_element_type=jnp.float32)
    x = x + bf[...] + embb[...]
    y1_o[...] = jnp.dot(x, W1l[...], preferred_element_type=jnp.float32)
    r1_o[...] = jnp.dot(x, W1r[...], preferred_element_type=jnp.float32) + b1l[...]


def _rdeg(degp):
    deg = degp[0, :, 0] + degp[1, :, 0]
    return 1.0 / jnp.clip(deg, 1.0, None)


def _mid_body(aggp0, aggp1, degp, r1, W2r, b2l, h1_o, r2_o):
    agg = aggp0[...] + aggp1[...]
    h1 = jnp.maximum(agg * _rdeg(degp)[:, None] + r1[...], 0.0)
    h1_o[...] = h1
    r2_o[...] = jnp.dot(h1, W2r[...], preferred_element_type=jnp.float32) + b2l[...]


def _final_body(aggp0, aggp1, degp, r2, W2l, out_o):
    agg = aggp0[...] + aggp1[...]
    mean = agg * _rdeg(degp)[:, None]
    out_o[...] = jnp.maximum(
        jnp.dot(mean, W2l[...], preferred_element_type=jnp.float32) + r2[...],
        0.0)


def _row_spec(w):
    return pl.BlockSpec((_BLK, w), lambda i: (i, 0))


def _full_spec(*shape):
    nd = len(shape)
    return pl.BlockSpec(shape, lambda i, _nd=nd: (0,) * _nd)


def _part_spec(w):
    return pl.BlockSpec((2, _BLK, w), lambda i: (0, i, 0))


def _tc_prep(xf, emb, Wf, bf, W1l, b1l, W1r):
    return pl.pallas_call(
        _prep_body,
        grid=(N // _BLK,),
        in_specs=[
            _row_spec(F_IN), _row_spec(D), _full_spec(F_IN, D),
            _full_spec(1, D), _full_spec(D, D), _full_spec(1, D),
            _full_spec(D, D),
        ],
        out_specs=[_row_spec(D), _row_spec(D)],
        out_shape=[
            jax.ShapeDtypeStruct((N, D), jnp.float32),
            jax.ShapeDtypeStruct((N, D), jnp.float32),
        ],
    )(xf, emb, Wf, bf, W1l, b1l, W1r)


def _tc_mid(aggp0, aggp1, degp, r1, W2r, b2l):
    return pl.pallas_call(
        _mid_body,
        grid=(N // _BLK,),
        in_specs=[
            _row_spec(D), _row_spec(D), _part_spec(1), _row_spec(D),
            _full_spec(D, D_OUT), _full_spec(1, D_OUT),
        ],
        out_specs=[_row_spec(D), _row_spec(D_OUT)],
        out_shape=[
            jax.ShapeDtypeStruct((N, D), jnp.float32),
            jax.ShapeDtypeStruct((N, D_OUT), jnp.float32),
        ],
    )(aggp0, aggp1, degp, r1, W2r, b2l)


def _tc_final(aggp0, aggp1, degp, r2, W2l):
    return pl.pallas_call(
        _final_body,
        grid=(N // _BLK,),
        in_specs=[_row_spec(D), _row_spec(D), _part_spec(1),
                  _row_spec(D_OUT), _full_spec(D, D_OUT)],
        out_specs=_row_spec(D_OUT),
        out_shape=jax.ShapeDtypeStruct((N, D_OUT), jnp.float32),
    )(aggp0, aggp1, degp, r2, W2l)


_OUTER = (NCHUNK - 2) // 3


def _make_sc_agg(with_deg):
    out_type = [jax.ShapeDtypeStruct((N, D), jnp.float32),
                jax.ShapeDtypeStruct((N, D), jnp.float32)]
    scratch = [
        pltpu.VMEM_SHARED((N, D), jnp.float32),
        pltpu.VMEM((EPW,), jnp.int32),
        pltpu.VMEM((3, K), jnp.int32),
        pltpu.VMEM((3, K, D), jnp.float32),
    ] + [pltpu.SemaphoreType.DMA] * 9
    if with_deg:
        out_type.append(jax.ShapeDtypeStruct((NC, N), jnp.float32))
        scratch += [pltpu.SemaphoreType.DMA] * 3
        scratch += [
            pltpu.VMEM_SHARED((N,), jnp.float32),
            pltpu.VMEM((K,), jnp.float32),
        ]

    @functools.partial(
        pl.kernel, mesh=_MESH, out_type=out_type, scratch_types=scratch)
    def agg_kernel(y_hbm, src_hbm, dst_hbm, z_hbm, zn_hbm, ones_hbm, *rest):
        if with_deg:
            (agg_out0, agg_out1, deg_out, acc, srcb, dslot, rows,
             g0, g1, g2, d0, d1, d2, s0, s1, s2,
             e0, e1, e2, dega, onesb) = rest
            dgsem = (e0, e1, e2)
        else:
            (agg_out0, agg_out1, acc, srcb, dslot, rows,
             g0, g1, g2, d0, d1, d2, s0, s1, s2) = rest
        gsem = (g0, g1, g2)
        dsem = (d0, d1, d2)
        ssem = (s0, s1, s2)
        c = lax.axis_index("c")
        s = lax.axis_index("s")
        wid = c * NS + s
        base = wid * EPW
        pltpu.async_copy(src_hbm.at[pl.ds(base, EPW)], srcb, g0)
        pltpu.async_copy(z_hbm, acc.at[pl.ds(s * RPT, RPT)], g1)
        if with_deg:
            @pl.when(s == 0)
            def _():
                pltpu.async_copy(zn_hbm, dega, d0).wait()
            pltpu.async_copy(ones_hbm, onesb, d1)
            pltpu.make_async_copy(ones_hbm, onesb, d1).wait()
        pltpu.make_async_copy(src_hbm.at[pl.ds(base, EPW)], srcb, g0).wait()
        pltpu.make_async_copy(z_hbm, acc.at[pl.ds(s * RPT, RPT)], g1).wait()
        plsc.subcore_barrier()

        def fire(j, slot):
            pltpu.async_copy(y_hbm.at[srcb.at[pl.ds(j * K, K)]],
                             rows.at[slot], gsem[slot])
            pltpu.async_copy(dst_hbm.at[pl.ds(base + j * K, K)],
                             dslot.at[slot], dsem[slot])

        def wait_scat(slot):
            pltpu.make_async_copy(rows.at[slot], acc.at[pl.ds(0, K)],
                                  ssem[slot]).wait()
            if with_deg:
                pltpu.make_async_copy(onesb, dega.at[pl.ds(0, K)],
                                      dgsem[slot]).wait()

        def visit(j, slot, first=False):
            nslot = (slot + 2) % 3
            pltpu.make_async_copy(y_hbm.at[pl.ds(0, K)], rows.at[slot],
                                  gsem[slot]).wait()
            pltpu.make_async_copy(dst_hbm.at[pl.ds(0, K)],
                                  dslot.at[slot], dsem[slot]).wait()
            pltpu.async_copy(rows.at[slot], acc.at[dslot.at[slot]],
                             ssem[slot], add=True)
            if with_deg:
                pltpu.async_copy(onesb, dega.at[dslot.at[slot]],
                                 dgsem[slot], add=True)
            if not first:
                wait_scat(nslot)

            @pl.when(j + 2 < NCHUNK)
            def _():
                fire(j + 2, nslot)

        fire(0, 0)
        fire(1, 1)

        visit(0, 0, first=True)
        visit(1, 1)
        visit(2, 2)

        def body3(g, carry):
            for b in range(3):
                j = 3 + g * 3 + b
                visit(j, b)
            return carry

        lax.fori_loop(0, _OUTER - 1, body3, 0)
        visit(NCHUNK - 2, (NCHUNK - 2) % 3)
        visit(NCHUNK - 1, (NCHUNK - 1) % 3)
        wait_scat((NCHUNK - 1) % 3)

        plsc.subcore_barrier()

        @pl.when((s < 15) & (c == 0))
        def _():
            pltpu.sync_copy(acc.at[pl.ds(s * 624, 624)],
                            agg_out0.at[pl.ds(s * 624, 624)])

        @pl.when((s == 15) & (c == 0))
        def _():
            pltpu.sync_copy(acc.at[pl.ds(9360, 640)],
                            agg_out0.at[pl.ds(9360, 640)])

        @pl.when((s < 15) & (c == 1))
        def _():
            pltpu.sync_copy(acc.at[pl.ds(s * 624, 624)],
                            agg_out1.at[pl.ds(s * 624, 624)])

        @pl.when((s == 15) & (c == 1))
        def _():
            pltpu.sync_copy(acc.at[pl.ds(9360, 640)],
                            agg_out1.at[pl.ds(9360, 640)])
        if with_deg:
            @pl.when(s == 0)
            def _():
                pltpu.sync_copy(dega, deg_out.at[c])

    return agg_kernel


_sc_agg1 = _make_sc_agg(with_deg=True)
_sc_agg2 = _make_sc_agg(with_deg=False)


def kernel(x_features, edge_index, W_feat, b_feat, emb, W1l, b1l, W1r,
           W2l, b2l, W2r):
    z128 = jnp.zeros((RPT, D), jnp.float32)
    zn = jnp.zeros((N,), jnp.float32)
    ones1 = jnp.ones((K,), jnp.float32)

    y1, r1 = _tc_prep(x_features, emb, W_feat, b_feat.reshape(1, D),
                      W1l, b1l.reshape(1, D), W1r)
    src_e, dst_e = _tc_edges(edge_index)
    p1a, p1b, degp = _sc_agg1(y1, src_e, dst_e, z128, zn, ones1)
    degp = degp.reshape(NC, N, 1)
    h1, r2 = _tc_mid(p1a, p1b, degp, r1, W2r, b2l.reshape(1, D_OUT))
    p2a, p2b = _sc_agg2(h1, src_e, dst_e, z128, zn, ones1)
    return _tc_final(p2a, p2b, degp, r2, W2l)
